# Initial kernel scaffold; baseline (speedup 1.0000x reference)
import jax
import jax.numpy as jnp
from jax.experimental import pallas as pl


def kernel(x, edge_index, W, b):
    raise NotImplementedError("write your pallas kernel here")


# speedup vs baseline: 10.0403x; 10.0403x over previous
import functools
import math

import jax
import jax.numpy as jnp
from jax import lax
from jax.experimental import pallas as pl
from jax.experimental.pallas import tpu as pltpu
from jax.experimental.pallas import tpu_sc as plsc

NC = 2
NS = 16
NW = NC * NS
LANES = 16
CHUNK = 128
DEGW = 16
BLK = 512


def _make_deg_kernel(NPAD, K):
    RPT = NPAD // NS

    mesh = plsc.VectorSubcoreMesh(
        core_axis_name="c", subcore_axis_name="s", num_cores=NC,
        num_subcores=NS)

    @functools.partial(
        pl.kernel,
        out_type=jax.ShapeDtypeStruct((NC * NPAD,), jnp.float32),
        mesh=mesh,
        scratch_types=[
            pltpu.VMEM((CHUNK,), jnp.int32),
            pltpu.VMEM((CHUNK,), jnp.float32),
            pltpu.VMEM((RPT,), jnp.float32),
            pltpu.VMEM_SHARED((NPAD,), jnp.float32),
        ],
    )
    def deg_kernel(dst_hbm, deg_hbm, di_v, ones_v, zb, deg_sp):
        cid = lax.axis_index("c")
        sid = lax.axis_index("s")
        wid = sid * NC + cid

        one16 = jnp.ones((LANES,), jnp.float32)
        zero16 = jnp.zeros((LANES,), jnp.float32)

        def fill_ones(r, _):
            ones_v[pl.ds(r * LANES, LANES)] = one16
            return 0
        lax.fori_loop(0, CHUNK // LANES, fill_ones, 0)

        def fill_zero(r, _):
            zb[pl.ds(r * LANES, LANES)] = zero16
            return 0
        lax.fori_loop(0, RPT // LANES, fill_zero, 0)

        pltpu.sync_copy(zb, deg_sp.at[pl.ds(sid * RPT, RPT)])
        plsc.subcore_barrier()

        def scatter(c, _):
            pltpu.sync_copy(dst_hbm.at[wid, c], di_v)
            pltpu.sync_copy(ones_v, deg_sp.at[di_v], add=True)
            return 0
        lax.fori_loop(0, K, scatter, 0)

        plsc.subcore_barrier()
        pltpu.sync_copy(deg_sp.at[pl.ds(sid * RPT, RPT)], zb)
        pltpu.sync_copy(zb, deg_hbm.at[pl.ds(cid * NPAD + sid * RPT, RPT)])

    return deg_kernel


def _make_agg_kernel(NPAD, D, K):
    RPT = NPAD // NS
    DRAIN = RPT // CHUNK

    mesh = plsc.VectorSubcoreMesh(
        core_axis_name="c", subcore_axis_name="s", num_cores=NC,
        num_subcores=NS)

    @functools.partial(
        pl.kernel,
        out_type=jax.ShapeDtypeStruct((NC * NPAD, D), jnp.float32),
        mesh=mesh,
        scratch_types=[
            pltpu.VMEM((CHUNK,), jnp.int32),
            pltpu.VMEM((CHUNK,), jnp.int32),
            pltpu.VMEM((CHUNK,), jnp.int32),
            pltpu.VMEM((CHUNK, D), jnp.float32),
            pltpu.VMEM((CHUNK, D), jnp.float32),
            pltpu.SemaphoreType.DMA,
            pltpu.SemaphoreType.DMA,
            pltpu.VMEM_SHARED((NPAD, D), jnp.float32),
        ],
    )
    def agg_kernel(y_hbm, src_hbm, dst_hbm, out_hbm,
                   si_a, si_b, di_v, buf_a, buf_b, sem_a, sem_b, acc_sp):
        cid = lax.axis_index("c")
        sid = lax.axis_index("s")
        wid = sid * NC + cid

        zero16 = jnp.zeros((LANES,), jnp.float32)

        def fill_zero(r, _):
            for k in range(D // LANES):
                buf_a[r, pl.ds(k * LANES, LANES)] = zero16
            return 0
        lax.fori_loop(0, CHUNK, fill_zero, 0)

        for j in range(DRAIN):
            pltpu.sync_copy(
                buf_a, acc_sp.at[pl.ds(sid * RPT + j * CHUNK, CHUNK)])

        plsc.subcore_barrier()

        def step(j, _):
            c0 = 2 * j
            c1 = 2 * j + 1
            pltpu.sync_copy(src_hbm.at[wid, c0], si_a)
            d0 = pltpu.async_copy(y_hbm.at[si_a], buf_a, sem_a)
            pltpu.sync_copy(src_hbm.at[wid, c1], si_b)
            d1 = pltpu.async_copy(y_hbm.at[si_b], buf_b, sem_b)

            d0.wait()
            pltpu.sync_copy(dst_hbm.at[wid, c0], di_v)
            pltpu.sync_copy(buf_a, acc_sp.at[di_v], add=True)

            d1.wait()
            pltpu.sync_copy(dst_hbm.at[wid, c1], di_v)
            pltpu.sync_copy(buf_b, acc_sp.at[di_v], add=True)
            return 0
        lax.fori_loop(0, K // 2, step, 0)

        plsc.subcore_barrier()
        for j in range(DRAIN):
            r0 = sid * RPT + j * CHUNK
            pltpu.sync_copy(acc_sp.at[pl.ds(r0, CHUNK)], buf_a)
            pltpu.sync_copy(buf_a, out_hbm.at[pl.ds(cid * NPAD + r0, CHUNK)])

    return agg_kernel


def _dense_y(x_ref, w_ref, deg0_ref, deg1_ref, y_ref, dinv_ref):
    deg = deg0_ref[...] + deg1_ref[...] + 1.0
    dinv = lax.rsqrt(deg)
    h = jnp.where(x_ref[...] >= 0.0, x_ref[...], 0.5 * x_ref[...])
    y = jnp.dot(h, w_ref[...], preferred_element_type=jnp.float32,
                precision=lax.Precision.HIGHEST)
    y_ref[...] = y * dinv
    dinv_ref[...] = dinv


def _dense_fin(p0_ref, p1_ref, y_ref, dinv_ref, b_ref, o_ref):
    agg = p0_ref[...] + p1_ref[...] + y_ref[...]
    o_ref[...] = agg * dinv_ref[...] + b_ref[...]


def kernel(x, edge_index, W, b):
    N, D = x.shape
    E = edge_index.shape[1]
    NPAD = ((N + BLK) // BLK) * BLK
    K = -(-E // (NW * CHUNK))
    K += K % 2
    EP = NW * K * CHUNK

    src = edge_index[0]
    dst = edge_index[1]
    pad = jnp.full((EP - E,), N, dtype=jnp.int32)
    src_slab = jnp.concatenate([src, pad]).reshape(NW, K, CHUNK)
    dst_slab = jnp.concatenate([dst, pad]).reshape(NW, K, CHUNK)
    x_ext = jnp.pad(x, ((0, NPAD - N), (0, 0)))

    deg = _make_deg_kernel(NPAD, K)(dst_slab)
    deg0 = deg[:NPAD].reshape(NPAD, 1)
    deg1 = deg[NPAD:].reshape(NPAD, 1)

    nblk = NPAD // BLK
    y_ext, dinv = pl.pallas_call(
        _dense_y,
        grid=(nblk,),
        in_specs=[
            pl.BlockSpec((BLK, D), lambda i: (i, 0)),
            pl.BlockSpec((D, D), lambda i: (0, 0)),
            pl.BlockSpec((BLK, 1), lambda i: (i, 0)),
            pl.BlockSpec((BLK, 1), lambda i: (i, 0)),
        ],
        out_specs=[
            pl.BlockSpec((BLK, D), lambda i: (i, 0)),
            pl.BlockSpec((BLK, 1), lambda i: (i, 0)),
        ],
        out_shape=[
            jax.ShapeDtypeStruct((NPAD, D), jnp.float32),
            jax.ShapeDtypeStruct((NPAD, 1), jnp.float32),
        ],
    )(x_ext, W, deg0, deg1)

    parts = _make_agg_kernel(NPAD, D, K)(y_ext, src_slab, dst_slab)
    p0 = parts[:NPAD]
    p1 = parts[NPAD:]

    out = pl.pallas_call(
        _dense_fin,
        grid=(nblk,),
        in_specs=[
            pl.BlockSpec((BLK, D), lambda i: (i, 0)),
            pl.BlockSpec((BLK, D), lambda i: (i, 0)),
            pl.BlockSpec((BLK, D), lambda i: (i, 0)),
            pl.BlockSpec((BLK, 1), lambda i: (i, 0)),
            pl.BlockSpec((1, D), lambda i: (0, 0)),
        ],
        out_specs=pl.BlockSpec((BLK, D), lambda i: (i, 0)),
        out_shape=jax.ShapeDtypeStruct((NPAD, D), jnp.float32),
    )(p0, p1, y_ext, dinv, b.reshape(1, D))

    return out[:N]


# speedup vs baseline: 13.0910x; 1.3038x over previous
import functools

import jax
import jax.numpy as jnp
from jax import lax
from jax.experimental import pallas as pl
from jax.experimental.pallas import tpu as pltpu
from jax.experimental.pallas import tpu_sc as plsc

NC = 2
NS = 16
NW = NC * NS
LANES = 16
CHUNK = 128
BLK = 512
SHIFT = 14
MASK = (1 << SHIFT) - 1


def _make_deg_kernel(NPAD, K):
    RPT = NPAD // NS

    mesh = plsc.VectorSubcoreMesh(
        core_axis_name="c", subcore_axis_name="s", num_cores=NC,
        num_subcores=NS)

    @functools.partial(
        pl.kernel,
        out_type=jax.ShapeDtypeStruct((NC * NPAD,), jnp.float32),
        mesh=mesh,
        scratch_types=[
            pltpu.VMEM((K, CHUNK), jnp.int32),
            pltpu.VMEM((CHUNK,), jnp.int32),
            pltpu.VMEM((CHUNK,), jnp.float32),
            pltpu.VMEM((RPT,), jnp.float32),
            pltpu.VMEM_SHARED((NPAD,), jnp.float32),
        ],
    )
    def deg_kernel(pk_hbm, deg_hbm, pk_v, di_v, ones_v, zb, deg_sp):
        cid = lax.axis_index("c")
        sid = lax.axis_index("s")
        wid = sid * NC + cid

        pltpu.sync_copy(pk_hbm.at[wid], pk_v)

        one16 = jnp.ones((LANES,), jnp.float32)
        zero16 = jnp.zeros((LANES,), jnp.float32)

        def fill_ones(r, _):
            ones_v[pl.ds(r * LANES, LANES)] = one16
            return 0
        lax.fori_loop(0, CHUNK // LANES, fill_ones, 0)

        def fill_zero(r, _):
            zb[pl.ds(r * LANES, LANES)] = zero16
            return 0
        lax.fori_loop(0, RPT // LANES, fill_zero, 0)

        pltpu.sync_copy(zb, deg_sp.at[pl.ds(sid * RPT, RPT)])
        plsc.subcore_barrier()

        def scatter(c, _):
            for k in range(CHUNK // LANES):
                v = pk_v[c, pl.ds(k * LANES, LANES)]
                di_v[pl.ds(k * LANES, LANES)] = v & MASK
            pltpu.sync_copy(ones_v, deg_sp.at[di_v], add=True)
            return 0
        lax.fori_loop(0, K, scatter, 0)

        plsc.subcore_barrier()
        pltpu.sync_copy(deg_sp.at[pl.ds(sid * RPT, RPT)], zb)
        pltpu.sync_copy(zb, deg_hbm.at[pl.ds(cid * NPAD + sid * RPT, RPT)])

    return deg_kernel


def _make_agg_kernel(NPAD, D, K):
    RPT = NPAD // NS
    DRAIN = RPT // CHUNK

    mesh = plsc.VectorSubcoreMesh(
        core_axis_name="c", subcore_axis_name="s", num_cores=NC,
        num_subcores=NS)

    @functools.partial(
        pl.kernel,
        out_type=jax.ShapeDtypeStruct((NC * NPAD, D), jnp.float32),
        mesh=mesh,
        scratch_types=[
            pltpu.VMEM((K, CHUNK), jnp.int32),
            pltpu.VMEM((CHUNK,), jnp.int32),
            pltpu.VMEM((CHUNK,), jnp.int32),
            pltpu.VMEM((CHUNK,), jnp.int32),
            pltpu.VMEM((CHUNK,), jnp.int32),
            pltpu.VMEM((CHUNK, D), jnp.float32),
            pltpu.VMEM((CHUNK, D), jnp.float32),
            pltpu.SemaphoreType.DMA,
            pltpu.SemaphoreType.DMA,
            pltpu.SemaphoreType.DMA,
            pltpu.SemaphoreType.DMA,
            pltpu.VMEM_SHARED((NPAD, D), jnp.float32),
        ],
    )
    def agg_kernel(y_hbm, pk_hbm, out_hbm,
                   pk_v, si_a, si_b, di_a, di_b, buf_a, buf_b,
                   ga, gb, sa, sb, acc_sp):
        cid = lax.axis_index("c")
        sid = lax.axis_index("s")
        wid = sid * NC + cid

        pltpu.sync_copy(pk_hbm.at[wid], pk_v)

        def unpack_src(c, out_ref):
            for k in range(CHUNK // LANES):
                v = pk_v[c, pl.ds(k * LANES, LANES)]
                out_ref[pl.ds(k * LANES, LANES)] = v >> SHIFT

        def unpack_dst(c, out_ref):
            for k in range(CHUNK // LANES):
                v = pk_v[c, pl.ds(k * LANES, LANES)]
                out_ref[pl.ds(k * LANES, LANES)] = v & MASK

        zero16 = jnp.zeros((LANES,), jnp.float32)

        def fill_zero(r, _):
            for k in range(D // LANES):
                buf_a[r, pl.ds(k * LANES, LANES)] = zero16
            return 0
        lax.fori_loop(0, CHUNK, fill_zero, 0)

        for j in range(DRAIN):
            pltpu.sync_copy(
                buf_a, acc_sp.at[pl.ds(sid * RPT + j * CHUNK, CHUNK)])

        unpack_src(0, si_a)
        pltpu.async_copy(y_hbm.at[si_a], buf_a, ga)
        unpack_src(1, si_b)
        pltpu.async_copy(y_hbm.at[si_b], buf_b, gb)
        plsc.subcore_barrier()

        def step(j, _):
            c0 = 2 * j
            c1 = 2 * j + 1

            pltpu.make_async_copy(y_hbm.at[si_a], buf_a, ga).wait()
            unpack_dst(c0, di_a)
            d_sa = pltpu.async_copy(buf_a, acc_sp.at[di_a], sa, add=True)

            pltpu.make_async_copy(y_hbm.at[si_b], buf_b, gb).wait()
            unpack_dst(c1, di_b)
            d_sb = pltpu.async_copy(buf_b, acc_sp.at[di_b], sb, add=True)

            @pl.when(c0 + 2 < K)
            def _():
                unpack_src(c0 + 2, si_a)
                d_sa.wait()
                pltpu.async_copy(y_hbm.at[si_a], buf_a, ga)

            @pl.when(c1 + 2 < K)
            def _():
                unpack_src(c1 + 2, si_b)
                d_sb.wait()
                pltpu.async_copy(y_hbm.at[si_b], buf_b, gb)
            return 0
        lax.fori_loop(0, K // 2, step, 0)

        pltpu.make_async_copy(buf_a, acc_sp.at[di_a], sa).wait()
        pltpu.make_async_copy(buf_b, acc_sp.at[di_b], sb).wait()

        plsc.subcore_barrier()
        for j in range(DRAIN):
            r0 = sid * RPT + j * CHUNK
            pltpu.sync_copy(acc_sp.at[pl.ds(r0, CHUNK)], buf_a)
            pltpu.sync_copy(buf_a, out_hbm.at[pl.ds(cid * NPAD + r0, CHUNK)])

    return agg_kernel


def _dense_y(x_ref, w_ref, deg0_ref, deg1_ref, y_ref, dinv_ref):
    deg = deg0_ref[...] + deg1_ref[...] + 1.0
    dinv = lax.rsqrt(deg)
    h = jnp.where(x_ref[...] >= 0.0, x_ref[...], 0.5 * x_ref[...])
    y = jnp.dot(h, w_ref[...], preferred_element_type=jnp.float32,
                precision=lax.Precision.HIGHEST)
    y_ref[...] = y * dinv
    dinv_ref[...] = dinv


def _dense_fin(p0_ref, p1_ref, y_ref, dinv_ref, b_ref, o_ref):
    agg = p0_ref[...] + p1_ref[...] + y_ref[...]
    o_ref[...] = agg * dinv_ref[...] + b_ref[...]


def kernel(x, edge_index, W, b):
    N, D = x.shape
    E = edge_index.shape[1]
    NPAD = ((N + BLK) // BLK) * BLK
    K = -(-E // (NW * CHUNK))
    K += K % 2
    EP = NW * K * CHUNK

    src = edge_index[0]
    dst = edge_index[1]
    packed = jnp.concatenate([
        (src << SHIFT) | dst,
        jnp.full((EP - E,), (N << SHIFT) | N, dtype=jnp.int32),
    ]).reshape(NW, K, CHUNK)
    x_ext = jnp.pad(x, ((0, NPAD - N), (0, 0)))

    deg = _make_deg_kernel(NPAD, K)(packed)
    deg0 = deg[:NPAD].reshape(NPAD, 1)
    deg1 = deg[NPAD:].reshape(NPAD, 1)

    nblk = NPAD // BLK
    y_ext, dinv = pl.pallas_call(
        _dense_y,
        grid=(nblk,),
        in_specs=[
            pl.BlockSpec((BLK, D), lambda i: (i, 0)),
            pl.BlockSpec((D, D), lambda i: (0, 0)),
            pl.BlockSpec((BLK, 1), lambda i: (i, 0)),
            pl.BlockSpec((BLK, 1), lambda i: (i, 0)),
        ],
        out_specs=[
            pl.BlockSpec((BLK, D), lambda i: (i, 0)),
            pl.BlockSpec((BLK, 1), lambda i: (i, 0)),
        ],
        out_shape=[
            jax.ShapeDtypeStruct((NPAD, D), jnp.float32),
            jax.ShapeDtypeStruct((NPAD, 1), jnp.float32),
        ],
    )(x_ext, W, deg0, deg1)

    parts = _make_agg_kernel(NPAD, D, K)(y_ext, packed)
    p0 = parts[:NPAD]
    p1 = parts[NPAD:]

    out = pl.pallas_call(
        _dense_fin,
        grid=(nblk,),
        in_specs=[
            pl.BlockSpec((BLK, D), lambda i: (i, 0)),
            pl.BlockSpec((BLK, D), lambda i: (i, 0)),
            pl.BlockSpec((BLK, D), lambda i: (i, 0)),
            pl.BlockSpec((BLK, 1), lambda i: (i, 0)),
            pl.BlockSpec((1, D), lambda i: (0, 0)),
        ],
        out_specs=pl.BlockSpec((BLK, D), lambda i: (i, 0)),
        out_shape=jax.ShapeDtypeStruct((NPAD, D), jnp.float32),
    )(p0, p1, y_ext, dinv, b.reshape(1, D))

    return out[:N]
